# baseline (speedup 1.0000x reference)
import functools

import jax
import jax.numpy as jnp
from jax import lax
from jax.experimental import pallas as pl
from jax.experimental.pallas import tpu as pltpu
from jax.experimental.pallas import tpu_sc as plsc

_B, _N, _D, _FH, _FW, _C = 1, 6, 59, 32, 88, 80
_PIX = _FH * _FW
_ROWS = _N * _D
_ROWS_PAD = 360
_NP = _ROWS * _PIX
_NX = _NY = 180
_NBINS = _NX * _NY
_DUMP = _NBINS
_CH = _C // 2
_ACC_ROWS = 32768
_ZROWS = _ACC_ROWS // 16

_IDX_ROWS = _NP // 128
_ROWS_HI = 487
_ROWS_LAST = _IDX_ROWS - 15 * _ROWS_HI


def _geom_body(pix_ref, par_ref, out_ref):
    xs = pix_ref[0:1, :]
    ys = pix_ref[1:2, :]
    P = par_ref

    def c(k):
        return P[:, k:k + 1]

    def bf(v):
        return v.astype(jnp.bfloat16).astype(jnp.float32)

    px = bf(xs - c(0))
    py = bf(ys - c(1))
    pz = bf(c(3) - c(2))
    q0 = (bf(c(4)) * px + bf(c(5)) * py) + bf(c(6)) * pz
    q1 = (bf(c(7)) * px + bf(c(8)) * py) + bf(c(9)) * pz
    q2 = (bf(c(10)) * px + bf(c(11)) * py) + bf(c(12)) * pz
    r0 = bf(q0 * q2)
    r1 = bf(q1 * q2)
    q2b = bf(q2)
    s0 = ((bf(c(13)) * r0 + bf(c(14)) * r1) + bf(c(15)) * q2b) + c(22)
    s1 = ((bf(c(16)) * r0 + bf(c(17)) * r1) + bf(c(18)) * q2b) + c(23)
    s2 = ((bf(c(19)) * r0 + bf(c(20)) * r1) + bf(c(21)) * q2b) + c(24)
    s0b, s1b, s2b = bf(s0), bf(s1), bf(s2)
    t0 = ((bf(c(25)) * s0b + bf(c(26)) * s1b) + bf(c(27)) * s2b) + c(34)
    t1 = ((bf(c(28)) * s0b + bf(c(29)) * s1b) + bf(c(30)) * s2b) + c(35)
    t2 = ((bf(c(31)) * s0b + bf(c(32)) * s1b) + bf(c(33)) * s2b) + c(36)
    g0 = (t0 - c(37)) / c(40)
    g1 = (t1 - c(38)) / c(41)
    g2 = (t2 - c(39)) / c(42)
    ix = g0.astype(jnp.int32)
    iy = g1.astype(jnp.int32)
    iz = g2.astype(jnp.int32)
    ex = jnp.where(ix < 0, ix + _NX, ix)
    ey = jnp.where(iy < 0, iy + _NY, iy)
    ez = jnp.where(iz < 0, iz + 1, iz)
    valid = ((ex >= 0) & (ex < _NX) & (ey >= 0) & (ey < _NY) & (ez == 0))
    out_ref[...] = jnp.where(valid, ex * _NY + ey, _DUMP)


def _compute_bins(pix, params):
    return pl.pallas_call(
        _geom_body,
        out_shape=jax.ShapeDtypeStruct((_ROWS_PAD, _PIX), jnp.int32),
        grid=(_ROWS_PAD // 8,),
        in_specs=[
            pl.BlockSpec((8, _PIX), lambda i: (0, 0)),
            pl.BlockSpec((8, 128), lambda i: (i, 0)),
        ],
        out_specs=pl.BlockSpec((8, _PIX), lambda i: (i, 0)),
    )(pix, params)


@functools.lru_cache(maxsize=1)
def _get_sc_scatter():
    mesh = plsc.VectorSubcoreMesh(core_axis_name="c", subcore_axis_name="s")

    nsup = 120
    npairs = nsup // 2

    @functools.partial(
        pl.kernel,
        out_type=jax.ShapeDtypeStruct((2 * _NBINS, _CH), jnp.float32),
        mesh=mesh,
        compiler_params=pltpu.CompilerParams(use_tc_tiling_on_sc=False),
        scratch_types=[
            pltpu.VMEM((2, 4, 128), jnp.int32),
            pltpu.VMEM((2, 512, _CH), jnp.float32),
            pltpu.VMEM_SHARED((_ACC_ROWS, _CH), jnp.float32),
            pltpu.SemaphoreType.DMA,
            pltpu.SemaphoreType.DMA,
            pltpu.SemaphoreType.DMA,
            pltpu.SemaphoreType.DMA,
        ],
    )
    def _sc_scatter(bins_hbm, x_hbm, zeros_hbm, out_hbm, idx_v, rows_v, acc,
                    sl0, sl1, ss0, ss1):
        core = lax.axis_index("c")
        sid = lax.axis_index("s")
        pltpu.sync_copy(zeros_hbm, acc.at[pl.ds(sid * _ZROWS, _ZROWS)])
        plsc.subcore_barrier()

        base_row = sid * _ROWS_HI
        nrows = jnp.where(sid < 15, _ROWS_HI, _ROWS_LAST)
        sem_l = (sl0, sl1)
        sem_s = (ss0, ss1)

        xoff = core * _NP

        def load_pair(slot, sup):
            r0 = base_row + sup * 4
            return (
                pltpu.async_copy(bins_hbm.at[pl.ds(r0, 4)], idx_v.at[slot],
                                 sem_l[slot]),
                pltpu.async_copy(
                    x_hbm.at[pl.ds(xoff + r0 * 128, 512)],
                    rows_v.at[slot], sem_l[slot]),
            )

        def wait_load(slot, sup):
            for c in load_pair_desc(slot, sup):
                c.wait()

        def load_pair_desc(slot, sup):
            r0 = base_row + sup * 4
            return (
                pltpu.make_async_copy(bins_hbm.at[pl.ds(r0, 4)],
                                      idx_v.at[slot], sem_l[slot]),
                pltpu.make_async_copy(
                    x_hbm.at[pl.ds(xoff + r0 * 128, 512)],
                    rows_v.at[slot], sem_l[slot]),
            )

        def fire_scatters(slot):
            for t in range(4):
                pltpu.async_copy(rows_v.at[slot, pl.ds(t * 128, 128)],
                                 acc.at[idx_v.at[slot, t]], sem_s[slot],
                                 add=True)

        def drain_scatters(slot):
            for t in range(4):
                pltpu.make_async_copy(rows_v.at[slot, pl.ds(t * 128, 128)],
                                      acc.at[idx_v.at[slot, t]],
                                      sem_s[slot]).wait()

        load_pair(0, 0)

        def body(p, carry):
            a = p * 2
            wait_load(0, a)
            fire_scatters(0)

            @pl.when(p > 0)
            def _():
                drain_scatters(1)

            load_pair(1, a + 1)
            drain_scatters(0)

            @pl.when(p < npairs - 1)
            def _():
                load_pair(0, a + 2)

            wait_load(1, a + 1)
            fire_scatters(1)
            return carry

        lax.fori_loop(0, npairs, body, 0)
        drain_scatters(1)

        def tail(i, carry):
            r = base_row + i
            pltpu.sync_copy(bins_hbm.at[pl.ds(r, 1)],
                            idx_v.at[0, pl.ds(0, 1)])
            pltpu.sync_copy(
                x_hbm.at[pl.ds(xoff + r * 128, 128)],
                rows_v.at[0, pl.ds(0, 128)])
            pltpu.sync_copy(rows_v.at[0, pl.ds(0, 128)],
                            acc.at[idx_v.at[0, 0]], add=True)
            return carry

        lax.fori_loop(nsup * 4, nrows, tail, 0)
        plsc.subcore_barrier()
        o0 = sid * (_NBINS // 16)
        pltpu.sync_copy(
            acc.at[pl.ds(o0, _NBINS // 16)],
            out_hbm.at[pl.ds(core * _NBINS + o0, _NBINS // 16)])

    return _sc_scatter


def kernel(x, camera_intrinsics, img_aug_matrix, camera2lidar, lidar_aug_matrix):
    f32 = jnp.float32
    intrins = camera_intrinsics[..., :3, :3]
    post_rots = img_aug_matrix[..., :3, :3]
    post_trans = img_aug_matrix[..., :3, 3]
    c2l_rots = camera2lidar[..., :3, :3]
    c2l_trans = camera2lidar[..., :3, 3]
    extra_rots = lidar_aug_matrix[..., :3, :3]
    extra_trans = lidar_aug_matrix[..., :3, 3]
    inv_post = jnp.linalg.inv(post_rots)
    combine = jnp.matmul(c2l_rots, jnp.linalg.inv(intrins))

    dxv = jnp.array([0.6, 0.6, 20.0], f32)
    bxv = jnp.array([-54.0 + 0.3, -54.0 + 0.3, -10.0 + 10.0], f32)
    off = bxv - dxv / 2.0

    pt = post_trans[0]
    ip = inv_post[0].reshape(6, 9)
    cb = combine[0].reshape(6, 9)
    tr = c2l_trans[0]
    er = jnp.broadcast_to(extra_rots[0].reshape(1, 9), (6, 9))
    et = jnp.broadcast_to(extra_trans[0].reshape(1, 3), (6, 3))
    ofb = jnp.broadcast_to(off.reshape(1, 3), (6, 3))
    dxb = jnp.broadcast_to(dxv.reshape(1, 3), (6, 3))
    cam = jnp.concatenate(
        [pt, jnp.zeros((6, 1), f32), ip, cb, tr, er, et, ofb, dxb], axis=1)
    n_idx = jnp.repeat(jnp.arange(_N), _D)
    ds_col = jnp.tile(jnp.arange(1.0, 60.0, 1.0, dtype=f32), _N)
    params354 = cam[n_idx].at[:, 3].set(ds_col)
    params = jnp.zeros((_ROWS_PAD, 128), f32).at[:_ROWS, :43].set(params354)

    xs_full = jnp.tile(jnp.linspace(0.0, _FW * 8.0 - 1.0, _FW, dtype=f32), _FH)
    ys_full = jnp.repeat(jnp.linspace(0.0, _FH * 8.0 - 1.0, _FH, dtype=f32), _FW)
    pix = jnp.zeros((8, _PIX), f32).at[0].set(xs_full).at[1].set(ys_full)

    bins2d = _compute_bins(pix, params)
    bins = bins2d.reshape(-1)[:_NP].reshape(_IDX_ROWS, 128)

    x2d = x.reshape(_NP, _C)
    xab = jnp.concatenate([x2d[:, :_CH], x2d[:, _CH:]], axis=0)
    zeros = jnp.zeros((_ZROWS, _CH), f32)
    acc = _get_sc_scatter()(bins, xab, zeros)

    out = acc.reshape(2, _NX, _NY, _CH).transpose(0, 3, 1, 2)
    return out.reshape(1, _C, _NX, _NY)


# speedup vs baseline: 2.3044x; 2.3044x over previous
import functools

import jax
import jax.numpy as jnp
from jax import lax
from jax.experimental import pallas as pl
from jax.experimental.pallas import tpu as pltpu
from jax.experimental.pallas import tpu_sc as plsc

_B, _N, _D, _FH, _FW, _C = 1, 6, 59, 32, 88, 80
_PIX = _FH * _FW
_ROWS = _N * _D
_ROWS_PAD = 360
_NP = _ROWS * _PIX
_NX = _NY = 180
_NBINS = _NX * _NY
_DUMP = _NBINS
_CH = _C // 2
_ACC_ROWS = 32768
_ZROWS = _ACC_ROWS // 16

_IDX_ROWS = _NP // 128
_ROWS_HI = 487
_ROWS_LAST = _IDX_ROWS - 15 * _ROWS_HI


def _geom_body(pix_ref, par_ref, out_ref):
    xs = pix_ref[0:1, :]
    ys = pix_ref[1:2, :]
    P = par_ref

    def c(k):
        return P[:, k:k + 1]

    def bf(v):
        return v.astype(jnp.bfloat16).astype(jnp.float32)

    px = bf(xs - c(0))
    py = bf(ys - c(1))
    pz = bf(c(3) - c(2))
    q0 = (bf(c(4)) * px + bf(c(5)) * py) + bf(c(6)) * pz
    q1 = (bf(c(7)) * px + bf(c(8)) * py) + bf(c(9)) * pz
    q2 = (bf(c(10)) * px + bf(c(11)) * py) + bf(c(12)) * pz
    r0 = bf(q0 * q2)
    r1 = bf(q1 * q2)
    q2b = bf(q2)
    s0 = ((bf(c(13)) * r0 + bf(c(14)) * r1) + bf(c(15)) * q2b) + c(22)
    s1 = ((bf(c(16)) * r0 + bf(c(17)) * r1) + bf(c(18)) * q2b) + c(23)
    s2 = ((bf(c(19)) * r0 + bf(c(20)) * r1) + bf(c(21)) * q2b) + c(24)
    s0b, s1b, s2b = bf(s0), bf(s1), bf(s2)
    t0 = ((bf(c(25)) * s0b + bf(c(26)) * s1b) + bf(c(27)) * s2b) + c(34)
    t1 = ((bf(c(28)) * s0b + bf(c(29)) * s1b) + bf(c(30)) * s2b) + c(35)
    t2 = ((bf(c(31)) * s0b + bf(c(32)) * s1b) + bf(c(33)) * s2b) + c(36)
    g0 = (t0 - c(37)) / c(40)
    g1 = (t1 - c(38)) / c(41)
    g2 = (t2 - c(39)) / c(42)
    ix = g0.astype(jnp.int32)
    iy = g1.astype(jnp.int32)
    iz = g2.astype(jnp.int32)
    ex = jnp.where(ix < 0, ix + _NX, ix)
    ey = jnp.where(iy < 0, iy + _NY, iy)
    ez = jnp.where(iz < 0, iz + 1, iz)
    valid = ((ex >= 0) & (ex < _NX) & (ey >= 0) & (ey < _NY) & (ez == 0))
    out_ref[...] = jnp.where(valid, ex * _NY + ey, _DUMP)


def _compute_bins(pix, params):
    return pl.pallas_call(
        _geom_body,
        out_shape=jax.ShapeDtypeStruct((_ROWS_PAD, _PIX), jnp.int32),
        grid=(_ROWS_PAD // 8,),
        in_specs=[
            pl.BlockSpec((8, _PIX), lambda i: (0, 0)),
            pl.BlockSpec((8, 128), lambda i: (i, 0)),
        ],
        out_specs=pl.BlockSpec((8, _PIX), lambda i: (i, 0)),
    )(pix, params)


@functools.lru_cache(maxsize=1)
def _get_sc_scatter():
    mesh = plsc.VectorSubcoreMesh(core_axis_name="c", subcore_axis_name="s")

    nsup = 120
    npairs = nsup // 2

    @functools.partial(
        pl.kernel,
        out_type=jax.ShapeDtypeStruct((2 * _NBINS, _CH), jnp.float32),
        mesh=mesh,
        compiler_params=pltpu.CompilerParams(use_tc_tiling_on_sc=False),
        scratch_types=[
            pltpu.VMEM((2, 4, 128), jnp.int32),
            pltpu.VMEM((2, 512, _CH), jnp.float32),
            pltpu.VMEM_SHARED((_ACC_ROWS, _CH), jnp.float32),
            pltpu.SemaphoreType.DMA,
            pltpu.SemaphoreType.DMA,
            pltpu.SemaphoreType.DMA,
            pltpu.SemaphoreType.DMA,
        ],
    )
    def _sc_scatter(bins_hbm, x_hbm, zeros_hbm, out_hbm, idx_v, rows_v, acc,
                    sl0, sl1, ss0, ss1):
        core = lax.axis_index("c")
        sid = lax.axis_index("s")
        pltpu.sync_copy(zeros_hbm, acc.at[pl.ds(sid * _ZROWS, _ZROWS)])
        plsc.subcore_barrier()

        base_row = sid * _ROWS_HI
        nrows = jnp.where(sid < 15, _ROWS_HI, _ROWS_LAST)
        sem_l = (sl0, sl1)
        sem_s = (ss0, ss1)

        cbase = core * _CH

        def load_pair(slot, sup):
            r0 = base_row + sup * 4
            return (
                pltpu.async_copy(bins_hbm.at[pl.ds(r0, 4)], idx_v.at[slot],
                                 sem_l[slot]),
                pltpu.async_copy(
                    x_hbm.at[pl.ds(r0 * 128, 512), pl.ds(cbase, _CH)],
                    rows_v.at[slot], sem_l[slot]),
            )

        def wait_load(slot, sup):
            for c in load_pair_desc(slot, sup):
                c.wait()

        def load_pair_desc(slot, sup):
            r0 = base_row + sup * 4
            return (
                pltpu.make_async_copy(bins_hbm.at[pl.ds(r0, 4)],
                                      idx_v.at[slot], sem_l[slot]),
                pltpu.make_async_copy(
                    x_hbm.at[pl.ds(r0 * 128, 512), pl.ds(cbase, _CH)],
                    rows_v.at[slot], sem_l[slot]),
            )

        def fire_scatters(slot):
            for t in range(4):
                pltpu.async_copy(rows_v.at[slot, pl.ds(t * 128, 128)],
                                 acc.at[idx_v.at[slot, t]], sem_s[slot],
                                 add=True)

        def drain_scatters(slot):
            for t in range(4):
                pltpu.make_async_copy(rows_v.at[slot, pl.ds(t * 128, 128)],
                                      acc.at[idx_v.at[slot, t]],
                                      sem_s[slot]).wait()

        load_pair(0, 0)

        def body(p, carry):
            a = p * 2
            wait_load(0, a)
            fire_scatters(0)

            @pl.when(p > 0)
            def _():
                drain_scatters(1)

            load_pair(1, a + 1)
            drain_scatters(0)

            @pl.when(p < npairs - 1)
            def _():
                load_pair(0, a + 2)

            wait_load(1, a + 1)
            fire_scatters(1)
            return carry

        lax.fori_loop(0, npairs, body, 0)
        drain_scatters(1)

        def tail(i, carry):
            r = base_row + i
            pltpu.sync_copy(bins_hbm.at[pl.ds(r, 1)],
                            idx_v.at[0, pl.ds(0, 1)])
            pltpu.sync_copy(
                x_hbm.at[pl.ds(r * 128, 128), pl.ds(cbase, _CH)],
                rows_v.at[0, pl.ds(0, 128)])
            pltpu.sync_copy(rows_v.at[0, pl.ds(0, 128)],
                            acc.at[idx_v.at[0, 0]], add=True)
            return carry

        lax.fori_loop(nsup * 4, nrows, tail, 0)
        plsc.subcore_barrier()
        o0 = sid * (_NBINS // 16)
        pltpu.sync_copy(
            acc.at[pl.ds(o0, _NBINS // 16)],
            out_hbm.at[pl.ds(core * _NBINS + o0, _NBINS // 16)])

    return _sc_scatter


def kernel(x, camera_intrinsics, img_aug_matrix, camera2lidar, lidar_aug_matrix):
    f32 = jnp.float32
    intrins = camera_intrinsics[..., :3, :3]
    post_rots = img_aug_matrix[..., :3, :3]
    post_trans = img_aug_matrix[..., :3, 3]
    c2l_rots = camera2lidar[..., :3, :3]
    c2l_trans = camera2lidar[..., :3, 3]
    extra_rots = lidar_aug_matrix[..., :3, :3]
    extra_trans = lidar_aug_matrix[..., :3, 3]
    inv_post = jnp.linalg.inv(post_rots)
    combine = jnp.matmul(c2l_rots, jnp.linalg.inv(intrins))

    dxv = jnp.array([0.6, 0.6, 20.0], f32)
    bxv = jnp.array([-54.0 + 0.3, -54.0 + 0.3, -10.0 + 10.0], f32)
    off = bxv - dxv / 2.0

    pt = post_trans[0]
    ip = inv_post[0].reshape(6, 9)
    cb = combine[0].reshape(6, 9)
    tr = c2l_trans[0]
    er = jnp.broadcast_to(extra_rots[0].reshape(1, 9), (6, 9))
    et = jnp.broadcast_to(extra_trans[0].reshape(1, 3), (6, 3))
    ofb = jnp.broadcast_to(off.reshape(1, 3), (6, 3))
    dxb = jnp.broadcast_to(dxv.reshape(1, 3), (6, 3))
    cam = jnp.concatenate(
        [pt, jnp.zeros((6, 1), f32), ip, cb, tr, er, et, ofb, dxb], axis=1)
    n_idx = jnp.repeat(jnp.arange(_N), _D)
    ds_col = jnp.tile(jnp.arange(1.0, 60.0, 1.0, dtype=f32), _N)
    params354 = cam[n_idx].at[:, 3].set(ds_col)
    params = jnp.zeros((_ROWS_PAD, 128), f32).at[:_ROWS, :43].set(params354)

    xs_full = jnp.tile(jnp.linspace(0.0, _FW * 8.0 - 1.0, _FW, dtype=f32), _FH)
    ys_full = jnp.repeat(jnp.linspace(0.0, _FH * 8.0 - 1.0, _FH, dtype=f32), _FW)
    pix = jnp.zeros((8, _PIX), f32).at[0].set(xs_full).at[1].set(ys_full)

    bins2d = _compute_bins(pix, params)
    bins = bins2d.reshape(-1)[:_NP].reshape(_IDX_ROWS, 128)

    x2d = x.reshape(_NP, _C)
    zeros = jnp.zeros((_ZROWS, _CH), f32)
    acc = _get_sc_scatter()(bins, x2d, zeros)

    out = acc.reshape(2, _NX, _NY, _CH).transpose(0, 3, 1, 2)
    return out.reshape(1, _C, _NX, _NY)
